# baseline (device time: 274097 ns/iter reference)
import os

import jax
import jax.numpy as jnp
from jax import lax
from jax.experimental import pallas as pl
from jax.experimental.pallas import tpu as pltpu

N_DEV = 4
M_PER = 2048
M_BLK = 1024
M_STEPS = M_PER // M_BLK
K = 8192
N_PER = 1024
K_BLK = 4096
K_STEPS = K // K_BLK
N_SLOTS = 2

_COMM = os.environ.get("DIAG_NO_COMM") != "1"


def kernel(x, w_mat):
    x = x.astype(jnp.bfloat16)
    w_mat = w_mat.astype(jnp.bfloat16)

    def _target(me, jj):
        off = jnp.where(
            jj == 0, 2, jnp.where(jj == 1, 1, jnp.where(jj == 2, 3, 0))
        )
        return lax.rem(me + off, N_DEV)

    def body(x_ref, w_ref, out_ref, acc_ref, y_buf, send_sems, recv_sems):
        jj = pl.program_id(0)
        m2 = pl.program_id(1)
        k = pl.program_id(2)
        me = lax.axis_index("i")
        j = _target(me, jj)

        if _COMM:
            @pl.when((jj == 0) & (m2 == 0) & (k == 0))
            def _entry_barrier():
                barrier_sem = pltpu.get_barrier_semaphore()
                for step in range(1, N_DEV):
                    pl.semaphore_signal(
                        barrier_sem, inc=1,
                        device_id=((me + step) % N_DEV,),
                        device_id_type=pl.DeviceIdType.MESH,
                    )
                pl.semaphore_wait(barrier_sem, N_DEV - 1)

        partial = jnp.dot(
            x_ref[...], w_ref[...], preferred_element_type=jnp.float32
        )

        @pl.when(k == 0)
        def _():
            acc_ref[...] = partial

        @pl.when(k > 0)
        def _():
            acc_ref[...] += partial

        dst_rows = pl.ds(me * M_PER + m2 * M_BLK, M_BLK)

        @pl.when(k == K_STEPS - 1)
        def _finish_half():
            slot = lax.rem(jj, N_SLOTS)
            if _COMM:
                @pl.when(jj >= N_SLOTS)
                def _reuse():
                    pltpu.make_async_remote_copy(
                        src_ref=y_buf.at[slot, pl.ds(m2 * M_BLK, M_BLK)],
                        dst_ref=out_ref.at[dst_rows, :],
                        send_sem=send_sems.at[jj - N_SLOTS, m2],
                        recv_sem=recv_sems.at[me, m2],
                        device_id=(j,),
                        device_id_type=pl.DeviceIdType.MESH,
                    ).wait_send()
            y_buf[slot, pl.ds(m2 * M_BLK, M_BLK)] = jnp.maximum(
                acc_ref[...], 0.0
            ).astype(jnp.bfloat16)

            @pl.when(jj != N_DEV - 1)
            def _send():
                if _COMM:
                    rdma = pltpu.make_async_remote_copy(
                        src_ref=y_buf.at[slot, pl.ds(m2 * M_BLK, M_BLK)],
                        dst_ref=out_ref.at[dst_rows, :],
                        send_sem=send_sems.at[jj, m2],
                        recv_sem=recv_sems.at[me, m2],
                        device_id=(j,),
                        device_id_type=pl.DeviceIdType.MESH,
                    )
                    rdma.start()

            @pl.when(jj == N_DEV - 1)
            def _local():
                pltpu.make_async_copy(
                    y_buf.at[slot, pl.ds(m2 * M_BLK, M_BLK)],
                    out_ref.at[dst_rows, :],
                    send_sems.at[N_DEV - 1, m2],
                ).start()

        @pl.when(
            (jj == N_DEV - 1) & (m2 == M_STEPS - 1) & (k == K_STEPS - 1)
        )
        def _drain():
            for jjp in (range(N_DEV - 2, N_DEV - 1) if _COMM else []):
                for h in range(M_STEPS):
                    pltpu.make_async_remote_copy(
                        src_ref=y_buf.at[jjp % N_SLOTS, pl.ds(h * M_BLK, M_BLK)],
                        dst_ref=out_ref.at[pl.ds(h * M_BLK, M_BLK), :],
                        send_sem=send_sems.at[jjp, h],
                        recv_sem=recv_sems.at[me, h],
                        device_id=(0,),
                        device_id_type=pl.DeviceIdType.MESH,
                    ).wait_send()
            for h in range(M_STEPS):
                pltpu.make_async_copy(
                    y_buf.at[(N_DEV - 1) % N_SLOTS, pl.ds(h * M_BLK, M_BLK)],
                    out_ref.at[pl.ds(me * M_PER + h * M_BLK, M_BLK), :],
                    send_sems.at[N_DEV - 1, h],
                ).wait()
            for step in (range(1, N_DEV) if _COMM else []):
                s = (me + step) % N_DEV
                for h in range(M_STEPS):
                    pltpu.make_async_remote_copy(
                        src_ref=y_buf.at[0, pl.ds(h * M_BLK, M_BLK)],
                        dst_ref=out_ref.at[
                            pl.ds(s * M_PER + h * M_BLK, M_BLK), :
                        ],
                        send_sem=send_sems.at[0, h],
                        recv_sem=recv_sems.at[s, h],
                        device_id=(s,),
                        device_id_type=pl.DeviceIdType.MESH,
                    ).wait_recv()

    return pl.pallas_call(
        body,
        grid=(N_DEV, M_STEPS, K_STEPS),
        in_specs=[
            pl.BlockSpec((M_BLK, K_BLK), lambda jj, m2, k: (m2, k)),
            pl.BlockSpec(
                (K_BLK, N_PER),
                lambda jj, m2, k: (k, _target(lax.axis_index("i"), jj)),
            ),
        ],
        out_specs=pl.BlockSpec(memory_space=pltpu.MemorySpace.HBM),
        out_shape=jax.ShapeDtypeStruct((N_DEV * M_PER, N_PER), jnp.bfloat16),
        scratch_shapes=[
            pltpu.VMEM((M_BLK, N_PER), jnp.float32),
            pltpu.VMEM((N_SLOTS, M_PER, N_PER), jnp.bfloat16),
            pltpu.SemaphoreType.DMA((N_DEV, M_STEPS)),
            pltpu.SemaphoreType.DMA((N_DEV, M_STEPS)),
        ],
        compiler_params=pltpu.CompilerParams(
            dimension_semantics=("arbitrary", "arbitrary", "arbitrary"),
            collective_id=0 if _COMM else None,
            vmem_limit_bytes=100 * 1024 * 1024,
        ),
    )(x, w_mat)


# device time: 206546 ns/iter; 1.3271x vs baseline; 1.3271x over previous
import os

import jax
import jax.numpy as jnp
from jax import lax
from jax.experimental import pallas as pl
from jax.experimental.pallas import tpu as pltpu

N_DEV = 4
M_PER = 2048
M_BLK = 1024
M_STEPS = M_PER // M_BLK
K = 8192
N_PER = 1024
K_BLK = 2048
K_STEPS = K // K_BLK
N_SLOTS = 2

_COMM = os.environ.get("DIAG_NO_COMM") != "1"


def kernel(x, w_mat):
    def _target(me, jj):
        off = jnp.where(
            jj == 0, 2, jnp.where(jj == 1, 1, jnp.where(jj == 2, 3, 0))
        )
        return lax.rem(me + off, N_DEV)

    def body(x_ref, w_ref, out_ref, acc_ref, y_buf, send_sems, recv_sems):
        jj = pl.program_id(0)
        m2 = pl.program_id(1)
        k = pl.program_id(2)
        me = lax.axis_index("i")
        j = _target(me, jj)

        if _COMM:
            @pl.when((jj == 0) & (m2 == 0) & (k == 0))
            def _entry_barrier():
                barrier_sem = pltpu.get_barrier_semaphore()
                for step in range(1, N_DEV):
                    pl.semaphore_signal(
                        barrier_sem, inc=1,
                        device_id=((me + step) % N_DEV,),
                        device_id_type=pl.DeviceIdType.MESH,
                    )
                pl.semaphore_wait(barrier_sem, N_DEV - 1)

        partial = jnp.dot(
            x_ref[...], w_ref[...], preferred_element_type=jnp.float32
        )

        @pl.when(k == 0)
        def _():
            acc_ref[...] = partial

        @pl.when(k > 0)
        def _():
            acc_ref[...] += partial

        dst_rows = pl.ds(me * M_PER + m2 * M_BLK, M_BLK)

        @pl.when(k == K_STEPS - 1)
        def _finish_half():
            slot = lax.rem(jj, N_SLOTS)
            if _COMM:
                @pl.when(jj >= N_SLOTS)
                def _reuse():
                    pltpu.make_async_remote_copy(
                        src_ref=y_buf.at[slot, pl.ds(m2 * M_BLK, M_BLK)],
                        dst_ref=out_ref.at[dst_rows, :],
                        send_sem=send_sems.at[jj - N_SLOTS, m2],
                        recv_sem=recv_sems.at[me, m2],
                        device_id=(j,),
                        device_id_type=pl.DeviceIdType.MESH,
                    ).wait_send()
            y_buf[slot, pl.ds(m2 * M_BLK, M_BLK)] = jnp.maximum(
                acc_ref[...], 0.0
            ).astype(jnp.bfloat16)

            @pl.when(jj != N_DEV - 1)
            def _send():
                if _COMM:
                    rdma = pltpu.make_async_remote_copy(
                        src_ref=y_buf.at[slot, pl.ds(m2 * M_BLK, M_BLK)],
                        dst_ref=out_ref.at[dst_rows, :],
                        send_sem=send_sems.at[jj, m2],
                        recv_sem=recv_sems.at[me, m2],
                        device_id=(j,),
                        device_id_type=pl.DeviceIdType.MESH,
                    )
                    rdma.start()

            @pl.when(jj == N_DEV - 1)
            def _local():
                pltpu.make_async_copy(
                    y_buf.at[slot, pl.ds(m2 * M_BLK, M_BLK)],
                    out_ref.at[dst_rows, :],
                    send_sems.at[N_DEV - 1, m2],
                ).start()

        @pl.when(
            (jj == N_DEV - 1) & (m2 == M_STEPS - 1) & (k == K_STEPS - 1)
        )
        def _drain():
            for jjp in (range(N_DEV - 2, N_DEV - 1) if _COMM else []):
                for h in range(M_STEPS):
                    pltpu.make_async_remote_copy(
                        src_ref=y_buf.at[jjp % N_SLOTS, pl.ds(h * M_BLK, M_BLK)],
                        dst_ref=out_ref.at[pl.ds(h * M_BLK, M_BLK), :],
                        send_sem=send_sems.at[jjp, h],
                        recv_sem=recv_sems.at[me, h],
                        device_id=(0,),
                        device_id_type=pl.DeviceIdType.MESH,
                    ).wait_send()
            for h in range(M_STEPS):
                pltpu.make_async_copy(
                    y_buf.at[(N_DEV - 1) % N_SLOTS, pl.ds(h * M_BLK, M_BLK)],
                    out_ref.at[pl.ds(me * M_PER + h * M_BLK, M_BLK), :],
                    send_sems.at[N_DEV - 1, h],
                ).wait()
            for step in (range(1, N_DEV) if _COMM else []):
                s = (me + step) % N_DEV
                for h in range(M_STEPS):
                    pltpu.make_async_remote_copy(
                        src_ref=y_buf.at[0, pl.ds(h * M_BLK, M_BLK)],
                        dst_ref=out_ref.at[
                            pl.ds(s * M_PER + h * M_BLK, M_BLK), :
                        ],
                        send_sem=send_sems.at[0, h],
                        recv_sem=recv_sems.at[s, h],
                        device_id=(s,),
                        device_id_type=pl.DeviceIdType.MESH,
                    ).wait_recv()

    return pl.pallas_call(
        body,
        grid=(N_DEV, M_STEPS, K_STEPS),
        in_specs=[
            pl.BlockSpec((M_BLK, K_BLK), lambda jj, m2, k: (m2, k)),
            pl.BlockSpec(
                (K_BLK, N_PER),
                lambda jj, m2, k: (k, _target(lax.axis_index("i"), jj)),
            ),
        ],
        out_specs=pl.BlockSpec(memory_space=pltpu.MemorySpace.HBM),
        out_shape=jax.ShapeDtypeStruct((N_DEV * M_PER, N_PER), jnp.bfloat16),
        scratch_shapes=[
            pltpu.VMEM((M_BLK, N_PER), jnp.float32),
            pltpu.VMEM((N_SLOTS, M_PER, N_PER), jnp.bfloat16),
            pltpu.SemaphoreType.DMA((N_DEV, M_STEPS)),
            pltpu.SemaphoreType.DMA((N_DEV, M_STEPS)),
        ],
        compiler_params=pltpu.CompilerParams(
            dimension_semantics=("arbitrary", "arbitrary", "arbitrary"),
            collective_id=0 if _COMM else None,
            vmem_limit_bytes=100 * 1024 * 1024,
        ),
    )(x, w_mat)
